# baseline (speedup 1.0000x reference)
import functools

import jax
import jax.numpy as jnp
from jax import lax
from jax.experimental import pallas as pl
from jax.experimental.pallas import tpu as pltpu
from jax.experimental.pallas import tpu_sc as plsc

BATCH = 16384
HIDDEN = 32
NROWS = 100000
NLANES = 100096
NTILECOLS = NLANES // 128
L = 16


def kernel(time_indices, table):
    info = plsc.get_sparse_core_info()
    nc, ns = info.num_cores, info.num_subcores
    half = HIDDEN // nc
    b_per_tec = BATCH // ns
    chunks_per_tec = (NTILECOLS + ns - 1) // ns

    mesh = plsc.VectorSubcoreMesh(core_axis_name="c", subcore_axis_name="s")

    iota16 = lambda: lax.iota(jnp.int32, L)

    @functools.partial(
        pl.kernel,
        mesh=mesh,
        out_type=jax.ShapeDtypeStruct((HIDDEN, BATCH), jnp.float32),
        compiler_params=pltpu.CompilerParams(
            use_tc_tiling_on_sc=True,
            needs_layout_passes=False,
            disable_bounds_checks=True,
            disable_semaphore_checks=True,
        ),
        scratch_types=[
            pltpu.VMEM_SHARED((NLANES // 8, 128), jnp.float32),
            pltpu.VMEM((half, 128), jnp.float32),
            pltpu.VMEM((half, 128), jnp.float32),
            pltpu.VMEM((b_per_tec,), jnp.int32),
            pltpu.VMEM((128,), jnp.int32),
            pltpu.VMEM((128,), jnp.int32),
            pltpu.VMEM((128, 128), jnp.float32),
            pltpu.VMEM((half, 128), jnp.float32),
            pltpu.SemaphoreType.DMA,
        ],
    )
    def gather_kernel(idx_hbm, tabt_hbm, outt_hbm, h_sp, blk_v, tp_v,
                      myidx_v, bid_v, rib_v, g_v, o_v, sem):
        sc = lax.axis_index("c")
        tec = lax.axis_index("s")
        crow = sc * half

        def do_tilecol(j, _):
            tc = tec * chunks_per_tec + j

            @pl.when(tc < NTILECOLS)
            def _():
                pltpu.sync_copy(
                    tabt_hbm.at[pl.ds(crow, half), pl.ds(tc * 128, 128)],
                    blk_v)

                def tp_row(h, _):
                    for s in range(8):
                        col = jnp.full((L,), h * 8 + s, jnp.int32)
                        tp_v[h, pl.ds(s * L, L)] = plsc.load_gather(
                            blk_v, [iota16(), col])
                    return _

                lax.fori_loop(0, half, tp_row, None)
                pltpu.sync_copy(tp_v, h_sp.at[pl.ds(tc * half, half)])
            return _

        lax.fori_loop(0, chunks_per_tec, do_tilecol, None)
        plsc.subcore_barrier()

        base = tec * b_per_tec
        pltpu.sync_copy(idx_hbm.at[pl.ds(base, b_per_tec)], myidx_v)
        for k in range(b_per_tec // 128):
            for q in range(8):
                v = myidx_v[pl.ds(k * 128 + q * L, L)]
                bid_v[pl.ds(q * L, L)] = lax.shift_right_logical(v, 3)
                rib_v[pl.ds(q * L, L)] = (v & 7) * L
            pltpu.async_copy(h_sp.at[bid_v], g_v, sem).wait()
            for q in range(8):
                row = q * L + iota16()
                rq = rib_v[pl.ds(q * L, L)]
                for c in range(half):
                    o_v[c, pl.ds(q * L, L)] = plsc.load_gather(
                        g_v, [row, rq + c])
            pltpu.sync_copy(
                o_v,
                outt_hbm.at[pl.ds(crow, half), pl.ds(base + k * 128, 128)])

    tab_t = table.T
    out_t = gather_kernel(time_indices.astype(jnp.int32), tab_t)
    return out_t.T


# speedup vs baseline: 1.3643x; 1.3643x over previous
import functools

import jax
import jax.numpy as jnp
from jax import lax
from jax.experimental import pallas as pl
from jax.experimental.pallas import tpu as pltpu
from jax.experimental.pallas import tpu_sc as plsc

BATCH = 16384
HIDDEN = 32
NROWS = 100000
NLANES = 100096
NTILECOLS = NLANES // 128
L = 16
CHB = 64


def kernel(time_indices, table):
    info = plsc.get_sparse_core_info()
    nc, ns = info.num_cores, info.num_subcores
    half = HIDDEN // nc
    b_per_tec = BATCH // ns
    chunks_per_tec = (NTILECOLS + ns - 1) // ns
    npairs = (chunks_per_tec + 1) // 2

    mesh = plsc.VectorSubcoreMesh(core_axis_name="c", subcore_axis_name="s")

    iota16 = lambda: lax.iota(jnp.int32, L)

    @functools.partial(
        pl.kernel,
        mesh=mesh,
        out_type=jax.ShapeDtypeStruct((HIDDEN, BATCH), jnp.float32),
        compiler_params=pltpu.CompilerParams(
            use_tc_tiling_on_sc=True,
            needs_layout_passes=False,
            disable_bounds_checks=True,
            disable_semaphore_checks=True,
        ),
        scratch_types=[
            pltpu.VMEM_SHARED((NLANES // 8, 128), jnp.float32),
            pltpu.VMEM((2, half, 128), jnp.float32),
            pltpu.VMEM((2, half, 128), jnp.float32),
            pltpu.VMEM((b_per_tec,), jnp.int32),
            pltpu.VMEM((2, CHB), jnp.int32),
            pltpu.VMEM((2, CHB), jnp.int32),
            pltpu.VMEM((2, CHB, 128), jnp.float32),
            pltpu.VMEM((half, 128), jnp.float32),
            pltpu.SemaphoreType.DMA,
            pltpu.SemaphoreType.DMA,
            pltpu.SemaphoreType.DMA,
            pltpu.SemaphoreType.DMA,
            pltpu.SemaphoreType.DMA,
        ],
    )
    def gather_kernel(idx_hbm, tabt_hbm, outt_hbm, h_sp, blk_v, tp_v,
                      myidx_v, bid_v, rib_v, g_v, o_v,
                      sem_in0, sem_in1, sem_out, sem_g0, sem_g1):
        sc = lax.axis_index("c")
        tec = lax.axis_index("s")
        crow = sc * half
        cbase = tec * chunks_per_tec
        sems_in = (sem_in0, sem_in1)

        def load_tc(j, b):
            pltpu.async_copy(
                tabt_hbm.at[pl.ds(crow, half),
                            pl.ds((cbase + j) * 128, 128)],
                blk_v.at[b], sems_in[b])

        def wait_in(b):
            pltpu.make_async_copy(
                tabt_hbm.at[pl.ds(crow, half), pl.ds(0, 128)],
                blk_v.at[b], sems_in[b]).wait()

        def do_chunk(j, b):
            wait_in(b)
            for h in range(half):
                for s in range(8):
                    col = jnp.full((L,), h * 8 + s, jnp.int32)
                    tp_v[b, h, pl.ds(s * L, L)] = plsc.load_gather(
                        blk_v.at[b], [iota16(), col])
            pltpu.async_copy(
                tp_v.at[b],
                h_sp.at[pl.ds((cbase + j) * half, half)], sem_out)

        nmine = jnp.clip(NTILECOLS - cbase, 0, chunks_per_tec)

        @pl.when(nmine > 0)
        def _():
            load_tc(0, 0)

            def pair(p, _):
                for b in range(2):
                    j = 2 * p + b

                    @pl.when(j + 1 < nmine)
                    def _():
                        load_tc(j + 1, 1 - b)

                    @pl.when(j < nmine)
                    def _():
                        do_chunk(j, b)
                return _

            lax.fori_loop(0, npairs, pair, None)

            def drain(j, _):
                pltpu.make_async_copy(
                    tabt_hbm.at[pl.ds(crow, half), pl.ds(0, 128)],
                    h_sp.at[pl.ds(0, half)], sem_out).wait()
                return _

            lax.fori_loop(0, nmine, drain, None)

        plsc.subcore_barrier()

        base = tec * b_per_tec
        pltpu.sync_copy(idx_hbm.at[pl.ds(base, b_per_tec)], myidx_v)
        nchunks = b_per_tec // CHB
        sems_g = (sem_g0, sem_g1)

        def prep(k):
            for q in range(CHB // L):
                v = myidx_v[pl.ds(k * CHB + q * L, L)]
                bid_v[k % 2, pl.ds(q * L, L)] = lax.shift_right_logical(v, 3)
                rib_v[k % 2, pl.ds(q * L, L)] = (v & 7) * L

        def fire(k):
            return pltpu.async_copy(
                h_sp.at[bid_v.at[k % 2]], g_v.at[k % 2], sems_g[k % 2])

        prep(0)
        pending = fire(0)
        for k in range(nchunks):
            b = k % 2
            nxt = None
            if k + 1 < nchunks:
                prep(k + 1)
                nxt = fire(k + 1)
            pending.wait()
            for q in range(CHB // L):
                row = q * L + iota16()
                rq = rib_v[b, pl.ds(q * L, L)]
                for c in range(half):
                    o_v[c, pl.ds(b * CHB + q * L, L)] = plsc.load_gather(
                        g_v.at[b], [row, rq + c])
            if b == 1:
                pltpu.sync_copy(
                    o_v,
                    outt_hbm.at[pl.ds(crow, half),
                                pl.ds(base + (k - 1) * CHB, 2 * CHB)])
            pending = nxt

    tab_t = table.T
    out_t = gather_kernel(time_indices.astype(jnp.int32), tab_t)
    return out_t.T


# speedup vs baseline: 2.1329x; 1.5634x over previous
import functools

import jax
import jax.numpy as jnp
from jax import lax
from jax.experimental import pallas as pl
from jax.experimental.pallas import tpu as pltpu
from jax.experimental.pallas import tpu_sc as plsc

BATCH = 16384
HIDDEN = 32
NROWS = 100000
NLANES = 100096
NTILECOLS = NLANES // 128
L = 16
CHB = 64


def kernel(time_indices, table):
    info = plsc.get_sparse_core_info()
    nc, ns = info.num_cores, info.num_subcores
    half = HIDDEN // nc
    b_per_tec = BATCH // ns
    chunks_per_tec = (NTILECOLS + ns - 1) // ns
    npairs = (chunks_per_tec + 1) // 2

    mesh = plsc.VectorSubcoreMesh(core_axis_name="c", subcore_axis_name="s")

    iota16 = lambda: lax.iota(jnp.int32, L)

    @functools.partial(
        pl.kernel,
        mesh=mesh,
        out_type=jax.ShapeDtypeStruct((HIDDEN, BATCH), jnp.float32),
        compiler_params=pltpu.CompilerParams(
            use_tc_tiling_on_sc=True,
            needs_layout_passes=False,
            disable_bounds_checks=True,
            disable_semaphore_checks=True,
        ),
        scratch_types=[
            pltpu.VMEM_SHARED((NLANES // 8, 128), jnp.float32),
            pltpu.VMEM((2, half, 128), jnp.float32),
            pltpu.VMEM((2, half, 128), jnp.float32),
            pltpu.VMEM((b_per_tec,), jnp.int32),
            pltpu.VMEM((2, CHB), jnp.int32),
            pltpu.VMEM((2, CHB), jnp.int32),
            pltpu.VMEM((2, CHB, 128), jnp.float32),
            pltpu.VMEM((half, 128), jnp.float32),
            pltpu.SemaphoreType.DMA,
            pltpu.SemaphoreType.DMA,
            pltpu.SemaphoreType.DMA,
            pltpu.SemaphoreType.DMA,
            pltpu.SemaphoreType.DMA,
        ],
    )
    def gather_kernel(idx_hbm, tabt_hbm, outt_hbm, h_sp, blk_v, tp_v,
                      myidx_v, bid_v, rib_v, g_v, o_v,
                      sem_in0, sem_in1, sem_out, sem_g0, sem_g1):
        sc = lax.axis_index("c")
        tec = lax.axis_index("s")
        crow = sc * half
        cbase = tec * chunks_per_tec
        sems_in = (sem_in0, sem_in1)

        def load_tc(j, b):
            pltpu.async_copy(
                tabt_hbm.at[pl.ds(crow, half),
                            pl.ds((cbase + j) * 128, 128)],
                blk_v.at[b], sems_in[b])

        def wait_in(b):
            pltpu.make_async_copy(
                tabt_hbm.at[pl.ds(crow, half), pl.ds(0, 128)],
                blk_v.at[b], sems_in[b]).wait()

        rowpat = lax.shift_right_logical(iota16(), 3)
        colpat = (iota16() & 7) * L

        def do_chunk(j, b):
            wait_in(b)
            for c in range(half):
                for h2 in range(8):
                    v = blk_v[b, c, pl.ds(h2 * L, L)]
                    plsc.store_scatter(
                        tp_v.at[b], [rowpat + 2 * h2, colpat + c], v)
            pltpu.async_copy(
                tp_v.at[b],
                h_sp.at[pl.ds((cbase + j) * half, half)], sem_out)

        nmine = jnp.clip(NTILECOLS - cbase, 0, chunks_per_tec)

        @pl.when(nmine > 0)
        def _():
            load_tc(0, 0)

            def pair(p, _):
                for b in range(2):
                    j = 2 * p + b

                    @pl.when(j + 1 < nmine)
                    def _():
                        load_tc(j + 1, 1 - b)

                    @pl.when(j < nmine)
                    def _():
                        do_chunk(j, b)
                return _

            lax.fori_loop(0, npairs, pair, None)

            def drain(j, _):
                pltpu.make_async_copy(
                    tabt_hbm.at[pl.ds(crow, half), pl.ds(0, 128)],
                    h_sp.at[pl.ds(0, half)], sem_out).wait()
                return _

            lax.fori_loop(0, nmine, drain, None)

        plsc.subcore_barrier()

        base = tec * b_per_tec
        pltpu.sync_copy(idx_hbm.at[pl.ds(base, b_per_tec)], myidx_v)
        nchunks = b_per_tec // CHB
        sems_g = (sem_g0, sem_g1)

        def prep(k):
            for q in range(CHB // L):
                v = myidx_v[pl.ds(k * CHB + q * L, L)]
                bid_v[k % 2, pl.ds(q * L, L)] = lax.shift_right_logical(v, 3)
                rib_v[k % 2, pl.ds(q * L, L)] = (v & 7) * L

        def fire(k):
            return pltpu.async_copy(
                h_sp.at[bid_v.at[k % 2]], g_v.at[k % 2], sems_g[k % 2])

        prep(0)
        pending = fire(0)
        for k in range(nchunks):
            b = k % 2
            nxt = None
            if k + 1 < nchunks:
                prep(k + 1)
                nxt = fire(k + 1)
            pending.wait()
            for q in range(CHB // L):
                row = q * L + iota16()
                rq = rib_v[b, pl.ds(q * L, L)]
                for c in range(half):
                    o_v[c, pl.ds(b * CHB + q * L, L)] = plsc.load_gather(
                        g_v.at[b], [row, rq + c])
            if b == 1:
                pltpu.sync_copy(
                    o_v,
                    outt_hbm.at[pl.ds(crow, half),
                                pl.ds(base + (k - 1) * CHB, 2 * CHB)])
            pending = nxt

    tab_t = table.T
    out_t = gather_kernel(time_indices.astype(jnp.int32), tab_t)
    return out_t.T


# speedup vs baseline: 2.3464x; 1.1001x over previous
import functools

import jax
import jax.numpy as jnp
from jax import lax
from jax.experimental import pallas as pl
from jax.experimental.pallas import tpu as pltpu
from jax.experimental.pallas import tpu_sc as plsc

BATCH = 16384
HIDDEN = 32
NROWS = 100000
NLANES = 100096
NTILECOLS = NLANES // 128
L = 16
CHB = 64


def kernel(time_indices, table):
    info = plsc.get_sparse_core_info()
    nc, ns = info.num_cores, info.num_subcores
    half = HIDDEN // nc
    b_per_tec = BATCH // ns
    chunks_per_tec = (NTILECOLS + ns - 1) // ns
    npairs = (chunks_per_tec + 1) // 2

    mesh = plsc.VectorSubcoreMesh(core_axis_name="c", subcore_axis_name="s")

    iota16 = lambda: lax.iota(jnp.int32, L)

    @functools.partial(
        pl.kernel,
        mesh=mesh,
        out_type=jax.ShapeDtypeStruct((HIDDEN, BATCH), jnp.float32),
        compiler_params=pltpu.CompilerParams(
            use_tc_tiling_on_sc=True,
            needs_layout_passes=False,
            disable_bounds_checks=True,
            disable_semaphore_checks=True,
            skip_device_barrier=True,
        ),
        scratch_types=[
            pltpu.VMEM_SHARED((NLANES // 8, 128), jnp.float32),
            pltpu.VMEM((2, half, 128), jnp.float32),
            pltpu.VMEM((2, half, 128), jnp.float32),
            pltpu.VMEM((b_per_tec,), jnp.int32),
            pltpu.VMEM((2, CHB), jnp.int32),
            pltpu.VMEM((2, CHB), jnp.int32),
            pltpu.VMEM((2, CHB, 128), jnp.float32),
            pltpu.VMEM((half, 128), jnp.float32),
            pltpu.SemaphoreType.DMA,
            pltpu.SemaphoreType.DMA,
            pltpu.SemaphoreType.DMA,
            pltpu.SemaphoreType.DMA,
            pltpu.SemaphoreType.DMA,
            pltpu.SemaphoreType.DMA,
        ],
    )
    def gather_kernel(idx_hbm, tabt_hbm, outt_hbm, h_sp, blk_v, tp_v,
                      myidx_v, bid_v, rib_v, g_v, o_v,
                      sem_in0, sem_in1, sem_out, sem_g0, sem_g1, sem_o):
        sc = lax.axis_index("c")
        tec = lax.axis_index("s")
        crow = sc * half
        cbase = tec * chunks_per_tec
        sems_in = (sem_in0, sem_in1)

        def load_tc(j, b):
            pltpu.async_copy(
                tabt_hbm.at[pl.ds(crow, half),
                            pl.ds((cbase + j) * 128, 128)],
                blk_v.at[b], sems_in[b])

        def wait_in(b):
            pltpu.make_async_copy(
                tabt_hbm.at[pl.ds(crow, half), pl.ds(0, 128)],
                blk_v.at[b], sems_in[b]).wait()

        rowpat = lax.shift_right_logical(iota16(), 3)
        colpat = (iota16() & 7) * L

        def do_chunk(j, b):
            wait_in(b)
            for c in range(half):
                for h2 in range(8):
                    v = blk_v[b, c, pl.ds(h2 * L, L)]
                    plsc.store_scatter(
                        tp_v.at[b], [rowpat + 2 * h2, colpat + c], v)
            pltpu.async_copy(
                tp_v.at[b],
                h_sp.at[pl.ds((cbase + j) * half, half)], sem_out)

        nmine = jnp.clip(NTILECOLS - cbase, 0, chunks_per_tec)

        @pl.when(nmine > 0)
        def _():
            load_tc(0, 0)

            def pair(p, _):
                for b in range(2):
                    j = 2 * p + b

                    @pl.when(j + 1 < nmine)
                    def _():
                        load_tc(j + 1, 1 - b)

                    @pl.when(j < nmine)
                    def _():
                        do_chunk(j, b)
                return _

            lax.fori_loop(0, npairs, pair, None)

            def drain(j, _):
                pltpu.make_async_copy(
                    tabt_hbm.at[pl.ds(crow, half), pl.ds(0, 128)],
                    h_sp.at[pl.ds(0, half)], sem_out).wait()
                return _

            lax.fori_loop(0, nmine, drain, None)

        plsc.subcore_barrier()

        base = tec * b_per_tec
        pltpu.sync_copy(idx_hbm.at[pl.ds(base, b_per_tec)], myidx_v)
        nchunks = b_per_tec // CHB
        sems_g = (sem_g0, sem_g1)

        def prep(k, slot):
            for q in range(CHB // L):
                v = myidx_v[pl.ds(k * CHB + q * L, L)]
                bid_v[slot, pl.ds(q * L, L)] = lax.shift_right_logical(v, 3)
                rib_v[slot, pl.ds(q * L, L)] = (v & 7) * L

        def fire(slot):
            pltpu.async_copy(h_sp.at[bid_v.at[slot]], g_v.at[slot],
                             sems_g[slot])

        def wait_g(slot):
            pltpu.make_async_copy(h_sp.at[pl.ds(0, CHB)], g_v.at[slot],
                                  sems_g[slot]).wait()

        def drain_o():
            pltpu.make_async_copy(
                tabt_hbm.at[pl.ds(crow, half), pl.ds(0, 128)],
                o_v, sem_o).wait()

        prep(0, 0)
        fire(0)

        def bpair(pp, _):
            for b in range(2):
                k = 2 * pp + b

                @pl.when(k + 1 < nchunks)
                def _():
                    prep(k + 1, 1 - b)
                    fire(1 - b)

                if b == 0:
                    @pl.when(k >= 2)
                    def _():
                        drain_o()

                wait_g(b)
                for q in range(CHB // L):
                    row = q * L + iota16()
                    rq = rib_v[b, pl.ds(q * L, L)]
                    for c in range(half):
                        o_v[c, pl.ds(b * CHB + q * L, L)] = plsc.load_gather(
                            g_v.at[b], [row, rq + c])
                if b == 1:
                    pltpu.async_copy(
                        o_v,
                        outt_hbm.at[pl.ds(crow, half),
                                    pl.ds(base + (k - 1) * CHB, 2 * CHB)],
                        sem_o)
            return _

        lax.fori_loop(0, nchunks // 2, bpair, None)
        drain_o()

    tab_t = table.T
    out_t = gather_kernel(time_indices.astype(jnp.int32), tab_t)
    return out_t.T
